# baseline (speedup 1.0000x reference)
import jax
import jax.numpy as jnp
from jax.experimental import pallas as pl

B, S, DIM = 8, 576, 256
K = 1024
NQ = 8
N = B * S
COMMIT_W = 0.25

ROW_BLK = 512
N_BLKS = N // ROW_BLK


def _rvq_body(x_ref, cb_ref, cb0_ref, cb1_ref, cb2_ref, cb3_ref, cbsq_ref,
              q_ref, *out_refs):
    idx_refs = out_refs[:NQ]
    counts_ref = out_refs[NQ]
    commit_ref = out_refs[NQ + 1]
    i = pl.program_id(0)

    @pl.when(i == 0)
    def _init():
        counts_ref[...] = jnp.zeros((NQ, K), jnp.float32)
        commit_ref[...] = jnp.zeros((NQ, 128), jnp.float32)

    x = x_ref[...]
    iota = jax.lax.broadcasted_iota(jnp.int32, (ROW_BLK, K), 1)
    r = x
    qsum = jnp.zeros_like(x)
    rsq = jnp.sum(r ** 2, axis=1, keepdims=True)
    for level in range(NQ):
        cb = cb_ref[level * K:(level + 1) * K, :]
        cbsq = cbsq_ref[level:level + 1, :]
        sc = jax.lax.dot_general(
            r, cb, (((1,), (1,)), ((), ())),
            preferred_element_type=jnp.float32)
        d = (rsq - 2.0 * sc) + cbsq
        minv = jnp.min(d, axis=1, keepdims=True)
        idx = jnp.min(jnp.where(d == minv, iota, K), axis=1)
        idx_refs[level][...] = idx.astype(jnp.int32)
        onehot = (iota == idx[:, None]).astype(jnp.float32)
        sl = slice(level * K, (level + 1) * K)
        oh16 = onehot.astype(jnp.bfloat16)
        dn = (((1,), (0,)), ((), ()))
        b0 = jax.lax.dot_general(oh16, cb0_ref[sl, :], dn,
                                 preferred_element_type=jnp.float32)
        b1 = jax.lax.dot_general(oh16, cb1_ref[sl, :], dn,
                                 preferred_element_type=jnp.float32)
        b2 = jax.lax.dot_general(oh16, cb2_ref[sl, :], dn,
                                 preferred_element_type=jnp.float32)
        b3 = jax.lax.dot_general(oh16, cb3_ref[sl, :], dn,
                                 preferred_element_type=jnp.float32)
        bits = ((b3.astype(jnp.uint32) << 24) | (b2.astype(jnp.uint32) << 16)
                | (b1.astype(jnp.uint32) << 8) | b0.astype(jnp.uint32))
        q = jax.lax.bitcast_convert_type(bits, jnp.float32)
        counts_ref[level:level + 1, :] += jnp.sum(onehot, axis=0,
                                                  keepdims=True)
        qsum = qsum + q
        r = r - q
        rsq = jnp.sum(r ** 2, axis=1, keepdims=True)
        commit_ref[level:level + 1, :] += jnp.sum(rsq)
    q_ref[...] = x + (qsum - x)


def _finalize_body(counts_ref, commit_ref, com_ref, perp_ref):
    counts = counts_ref[...]
    p = counts / N
    ent = jnp.sum(p * jnp.log(p + 1e-10), axis=1, keepdims=True)
    perps = jnp.exp(-ent)
    perp_ref[...] = jnp.full((8, 128), jnp.sum(perps) / NQ)
    commit = jnp.sum(commit_ref[...][:, 0:1]) / (N * DIM)
    com_ref[...] = jnp.full((8, 128), commit * COMMIT_W)


def kernel(x, codebooks):
    x_flat = x.reshape(N, DIM)
    cb_flat = codebooks.reshape(NQ * K, DIM)
    cb_bits = jax.lax.bitcast_convert_type(cb_flat, jnp.uint32)
    chunks = [((cb_bits >> (8 * j)) & 0xFF).astype(jnp.bfloat16)
              for j in range(4)]
    cbsq_all = jnp.sum(codebooks ** 2, axis=-1)

    outs = pl.pallas_call(
        _rvq_body,
        grid=(N_BLKS,),
        in_specs=[
            pl.BlockSpec((ROW_BLK, DIM), lambda i: (i, 0)),
            pl.BlockSpec((NQ * K, DIM), lambda i: (0, 0)),
            pl.BlockSpec((NQ * K, DIM), lambda i: (0, 0)),
            pl.BlockSpec((NQ * K, DIM), lambda i: (0, 0)),
            pl.BlockSpec((NQ * K, DIM), lambda i: (0, 0)),
            pl.BlockSpec((NQ * K, DIM), lambda i: (0, 0)),
            pl.BlockSpec((NQ, K), lambda i: (0, 0)),
        ],
        out_specs=[pl.BlockSpec((ROW_BLK, DIM), lambda i: (i, 0))]
        + [pl.BlockSpec((ROW_BLK,), lambda i: (i,)) for _ in range(NQ)]
        + [
            pl.BlockSpec((NQ, K), lambda i: (0, 0)),
            pl.BlockSpec((NQ, 128), lambda i: (0, 0)),
        ],
        out_shape=[jax.ShapeDtypeStruct((N, DIM), jnp.float32)]
        + [jax.ShapeDtypeStruct((N,), jnp.int32) for _ in range(NQ)]
        + [
            jax.ShapeDtypeStruct((NQ, K), jnp.float32),
            jax.ShapeDtypeStruct((NQ, 128), jnp.float32),
        ],
    )(x_flat, cb_flat, chunks[0], chunks[1], chunks[2], chunks[3], cbsq_all)

    quantized = outs[0]
    idx_list = outs[1:1 + NQ]
    counts, commit_acc = outs[1 + NQ], outs[2 + NQ]

    com, perp = pl.pallas_call(
        _finalize_body,
        out_shape=[
            jax.ShapeDtypeStruct((8, 128), jnp.float32),
            jax.ShapeDtypeStruct((8, 128), jnp.float32),
        ],
    )(counts, commit_acc)

    indices_out = jnp.stack(idx_list, axis=-1).reshape(B, S, NQ)
    quantized_out = quantized.reshape(B, S, DIM)
    return (quantized_out, indices_out, com[0, 0], perp[0, 0])


# speedup vs baseline: 1.0754x; 1.0754x over previous
import jax
import jax.numpy as jnp
from jax.experimental import pallas as pl

B, S, DIM = 8, 576, 256
K = 1024
NQ = 8
N = B * S
COMMIT_W = 0.25

ROW_BLK = 512
N_BLKS = N // ROW_BLK


def _rvq_body(x_ref, cb_ref, cb0_ref, cb1_ref, cb2_ref, cb3_ref, cbsq_ref,
              q_ref, *out_refs):
    idx_refs = out_refs[:NQ]
    counts_ref = out_refs[NQ]
    commit_ref = out_refs[NQ + 1]
    i = pl.program_id(0)

    @pl.when(i == 0)
    def _init():
        counts_ref[...] = jnp.zeros((NQ, K), jnp.float32)
        commit_ref[...] = jnp.zeros((NQ, 128), jnp.float32)

    x = x_ref[...]
    iota = jax.lax.broadcasted_iota(jnp.int32, (ROW_BLK, K), 1)
    r = x
    qsum = jnp.zeros_like(x)
    rsq = jnp.sum(r ** 2, axis=1, keepdims=True)
    for level in range(NQ):
        cb = cb_ref[level * K:(level + 1) * K, :]
        cbsq = cbsq_ref[level:level + 1, :]
        sc = jax.lax.dot_general(
            r, cb, (((1,), (1,)), ((), ())),
            preferred_element_type=jnp.float32)
        d = (rsq - 2.0 * sc) + cbsq
        idx = jnp.argmin(d, axis=1).astype(jnp.int32)
        idx_refs[level][...] = idx
        sl = slice(level * K, (level + 1) * K)
        oh16 = (iota == idx[:, None]).astype(jnp.bfloat16)
        dn = (((1,), (0,)), ((), ()))
        b0 = jax.lax.dot_general(oh16, cb0_ref[sl, :], dn,
                                 preferred_element_type=jnp.float32)
        b1 = jax.lax.dot_general(oh16, cb1_ref[sl, :], dn,
                                 preferred_element_type=jnp.float32)
        b2 = jax.lax.dot_general(oh16, cb2_ref[sl, :], dn,
                                 preferred_element_type=jnp.float32)
        b3 = jax.lax.dot_general(oh16, cb3_ref[sl, :], dn,
                                 preferred_element_type=jnp.float32)
        hi = b3 * 256.0 + b2
        lo = b1 * 256.0 + b0
        bits = (hi.astype(jnp.uint32) << 16) | lo.astype(jnp.uint32)
        q = jax.lax.bitcast_convert_type(bits, jnp.float32)
        counts_ref[level:level + 1, :] += jax.lax.dot_general(
            jnp.ones((1, ROW_BLK), jnp.bfloat16), oh16,
            (((1,), (0,)), ((), ())), preferred_element_type=jnp.float32)
        qsum = qsum + q
        r = r - q
        rsq = jnp.sum(r ** 2, axis=1, keepdims=True)
        commit_ref[level:level + 1, :] += jnp.sum(rsq)
    q_ref[...] = x + (qsum - x)


def _finalize_body(counts_ref, commit_ref, com_ref, perp_ref):
    counts = counts_ref[...]
    p = counts / N
    ent = jnp.sum(p * jnp.log(p + 1e-10), axis=1, keepdims=True)
    perps = jnp.exp(-ent)
    perp_ref[...] = jnp.full((8, 128), jnp.sum(perps) / NQ)
    commit = jnp.sum(commit_ref[...][:, 0:1]) / (N * DIM)
    com_ref[...] = jnp.full((8, 128), commit * COMMIT_W)


def kernel(x, codebooks):
    x_flat = x.reshape(N, DIM)
    cb_flat = codebooks.reshape(NQ * K, DIM)
    cb_bits = jax.lax.bitcast_convert_type(cb_flat, jnp.uint32)
    chunks = [((cb_bits >> (8 * j)) & 0xFF).astype(jnp.bfloat16)
              for j in range(4)]
    cbsq_all = jnp.sum(codebooks ** 2, axis=-1)

    outs = pl.pallas_call(
        _rvq_body,
        grid=(N_BLKS,),
        in_specs=[
            pl.BlockSpec((ROW_BLK, DIM), lambda i: (i, 0)),
            pl.BlockSpec((NQ * K, DIM), lambda i: (0, 0)),
            pl.BlockSpec((NQ * K, DIM), lambda i: (0, 0)),
            pl.BlockSpec((NQ * K, DIM), lambda i: (0, 0)),
            pl.BlockSpec((NQ * K, DIM), lambda i: (0, 0)),
            pl.BlockSpec((NQ * K, DIM), lambda i: (0, 0)),
            pl.BlockSpec((NQ, K), lambda i: (0, 0)),
        ],
        out_specs=[pl.BlockSpec((ROW_BLK, DIM), lambda i: (i, 0))]
        + [pl.BlockSpec((ROW_BLK,), lambda i: (i,)) for _ in range(NQ)]
        + [
            pl.BlockSpec((NQ, K), lambda i: (0, 0)),
            pl.BlockSpec((NQ, 128), lambda i: (0, 0)),
        ],
        out_shape=[jax.ShapeDtypeStruct((N, DIM), jnp.float32)]
        + [jax.ShapeDtypeStruct((N,), jnp.int32) for _ in range(NQ)]
        + [
            jax.ShapeDtypeStruct((NQ, K), jnp.float32),
            jax.ShapeDtypeStruct((NQ, 128), jnp.float32),
        ],
    )(x_flat, cb_flat, chunks[0], chunks[1], chunks[2], chunks[3], cbsq_all)

    quantized = outs[0]
    idx_list = outs[1:1 + NQ]
    counts, commit_acc = outs[1 + NQ], outs[2 + NQ]

    com, perp = pl.pallas_call(
        _finalize_body,
        out_shape=[
            jax.ShapeDtypeStruct((8, 128), jnp.float32),
            jax.ShapeDtypeStruct((8, 128), jnp.float32),
        ],
    )(counts, commit_acc)

    indices_out = jnp.stack(idx_list, axis=-1).reshape(B, S, NQ)
    quantized_out = quantized.reshape(B, S, DIM)
    return (quantized_out, indices_out, com[0, 0], perp[0, 0])
